# baseline (speedup 1.0000x reference)
import functools

import jax
import jax.numpy as jnp
from jax import lax
from jax.experimental import pallas as pl
from jax.experimental.pallas import tpu as pltpu
from jax.experimental.pallas import tpu_sc as plsc

_NC = 2
_NS = 16
_CHUNK = 128
_G = 64


def _sc_mesh():
    return plsc.VectorSubcoreMesh(core_axis_name="c", subcore_axis_name="s")


def _make_sc_pass(n, e, d, gather):
    erows = e // _CHUNK
    per_sc = erows // _NC
    q = per_sc // _NS
    r = per_sc % _NS
    nb = next(b for b in (6, 5, 4, 3, 2, 1) if q % b == 0)
    ng = q // nb
    rpt = n // _NS

    scratch = []
    if gather:
        scratch += [pltpu.VMEM((q, _CHUNK), jnp.int32),
                    pltpu.VMEM((1, _CHUNK), jnp.int32)]
    scratch += [
        pltpu.VMEM((q, _CHUNK), jnp.int32),
        pltpu.VMEM((1, _CHUNK), jnp.int32),
    ]
    if gather:
        scratch += [pltpu.VMEM((nb, _CHUNK, d), jnp.float32),
                    pltpu.VMEM_SHARED((n, d), jnp.float32)]
    else:
        scratch += [pltpu.VMEM((_CHUNK,), jnp.float32),
                    pltpu.VMEM_SHARED((n,), jnp.float32)]
    scratch += [pltpu.SemaphoreType.DMA]
    scratch += [pltpu.SemaphoreType.DMA] * (2 * nb if gather else nb)

    out_type = jax.ShapeDtypeStruct(
        (_NC * n, d) if gather else (_NC * n,), jnp.float32)
    params = pltpu.CompilerParams(use_tc_tiling_on_sc=False)

    if gather:
        @functools.partial(pl.kernel, out_type=out_type, mesh=_sc_mesh(),
                           scratch_types=scratch, compiler_params=params)
        def sc_pass(srcR, dstR, vals, zeros, out_hbm,
                    srcb, srcx, dstb, dstx, rows, agg, isem, *sems):
            gsem, ssem = sems[:nb], sems[nb:]
            cid = lax.axis_index("c")
            sid = lax.axis_index("s")
            row0 = sid * rpt
            er0 = cid * per_sc + sid * q
            xrow = cid * per_sc + _NS * q + sid
            pltpu.async_copy(srcR.at[pl.ds(er0, q), :], srcb, isem)
            pltpu.async_copy(dstR.at[pl.ds(er0, q), :], dstb, isem)

            @pl.when(sid < r)
            def _():
                pltpu.async_copy(srcR.at[pl.ds(xrow, 1), :], srcx, isem)
                pltpu.async_copy(dstR.at[pl.ds(xrow, 1), :], dstx, isem)

            @pl.when(cid == 0)
            def _():
                pltpu.sync_copy(vals.at[pl.ds(row0, rpt), :],
                                agg.at[pl.ds(row0, rpt), :])

            @pl.when(cid != 0)
            def _():
                pltpu.sync_copy(zeros, agg.at[pl.ds(row0, rpt), :])
            pltpu.make_async_copy(srcR.at[pl.ds(er0, q), :], srcb, isem).wait()
            pltpu.make_async_copy(dstR.at[pl.ds(er0, q), :], dstb, isem).wait()

            @pl.when(sid < r)
            def _():
                pltpu.make_async_copy(srcR.at[pl.ds(xrow, 1), :], srcx,
                                      isem).wait()
                pltpu.make_async_copy(dstR.at[pl.ds(xrow, 1), :], dstx,
                                      isem).wait()

            for u in range(nb):
                pltpu.async_copy(vals.at[srcb.at[u]], rows.at[u], gsem[u])
            plsc.subcore_barrier()

            def group(jj, carry):
                for u in range(nb):
                    pltpu.make_async_copy(vals.at[srcb.at[jj * nb + u]],
                                          rows.at[u], gsem[u]).wait()
                    pltpu.async_copy(rows.at[u], agg.at[dstb.at[jj * nb + u]],
                                     ssem[u], add=True)

                @pl.when(jj < ng - 1)
                def _():
                    for u in range(nb):
                        pltpu.make_async_copy(
                            rows.at[u], agg.at[dstb.at[jj * nb + u]],
                            ssem[u]).wait()
                        pltpu.async_copy(vals.at[srcb.at[(jj + 1) * nb + u]],
                                         rows.at[u], gsem[u])
                return carry

            lax.fori_loop(0, ng, group, 0)
            for u in range(nb):
                pltpu.make_async_copy(rows.at[u],
                                      agg.at[dstb.at[(ng - 1) * nb + u]],
                                      ssem[u]).wait()

            @pl.when(sid < r)
            def _():
                pltpu.async_copy(vals.at[srcx.at[0]], rows.at[0],
                                 gsem[0]).wait()
                pltpu.sync_copy(rows.at[0], agg.at[dstx.at[0]], add=True)

            plsc.subcore_barrier()
            pltpu.sync_copy(agg.at[pl.ds(row0, rpt), :],
                            out_hbm.at[pl.ds(cid * n + row0, rpt), :])
    else:
        @functools.partial(pl.kernel, out_type=out_type, mesh=_sc_mesh(),
                           scratch_types=scratch, compiler_params=params)
        def sc_pass(dstR, ones_hbm, zeros_hbm, out_hbm,
                    dstb, dstx, ones_v, agg, isem, *ssem):
            cid = lax.axis_index("c")
            sid = lax.axis_index("s")
            row0 = sid * rpt
            er0 = cid * per_sc + sid * q
            xrow = cid * per_sc + _NS * q + sid
            pltpu.async_copy(dstR.at[pl.ds(er0, q), :], dstb, isem)

            @pl.when(sid < r)
            def _():
                pltpu.async_copy(dstR.at[pl.ds(xrow, 1), :], dstx, isem)

            pltpu.sync_copy(ones_hbm, ones_v)
            pltpu.sync_copy(zeros_hbm, agg.at[pl.ds(row0, rpt)])
            pltpu.make_async_copy(dstR.at[pl.ds(er0, q), :], dstb, isem).wait()

            @pl.when(sid < r)
            def _():
                pltpu.make_async_copy(dstR.at[pl.ds(xrow, 1), :], dstx,
                                      isem).wait()

            plsc.subcore_barrier()

            def group(jj, carry):
                @pl.when(jj > 0)
                def _():
                    for u in range(nb):
                        pltpu.make_async_copy(
                            ones_v, agg.at[dstb.at[(jj - 1) * nb + u]],
                            ssem[u]).wait()
                for u in range(nb):
                    pltpu.async_copy(ones_v, agg.at[dstb.at[jj * nb + u]],
                                     ssem[u], add=True)
                return carry

            lax.fori_loop(0, ng, group, 0)
            for u in range(nb):
                pltpu.make_async_copy(ones_v,
                                      agg.at[dstb.at[(ng - 1) * nb + u]],
                                      ssem[u]).wait()

            @pl.when(sid < r)
            def _():
                pltpu.sync_copy(ones_v, agg.at[dstx.at[0]], add=True)

            plsc.subcore_barrier()
            pltpu.sync_copy(agg.at[pl.ds(row0, rpt)],
                            out_hbm.at[pl.ds(cid * n + row0, rpt)])

    return sc_pass


def _make_sc_pool(n, e):
    erows = e // _CHUNK
    per_sc = erows // _NC
    q = per_sc // _NS
    r = per_sc % _NS
    npw = -(-n // (_NC * _NS * 16)) * 16
    nchunks = npw // 16

    scratch = [
        pltpu.VMEM((q, _CHUNK), jnp.int32),
        pltpu.VMEM((q, _CHUNK), jnp.int32),
        pltpu.VMEM((1, _CHUNK), jnp.int32),
        pltpu.VMEM((1, _CHUNK), jnp.int32),
        pltpu.VMEM((n,), jnp.float32),
        pltpu.VMEM((n,), jnp.float32),
        pltpu.VMEM((n,), jnp.int32),
        pltpu.VMEM((16, _G), jnp.float32),
        pltpu.VMEM((16, _G), jnp.float32),
        pltpu.SemaphoreType.DMA,
    ]

    @functools.partial(
        pl.kernel,
        out_type=jax.ShapeDtypeStruct((2 * _NC * _NS * 16, _G), jnp.float32),
        mesh=_sc_mesh(),
        scratch_types=scratch,
        compiler_params=pltpu.CompilerParams(use_tc_tiling_on_sc=False,
                                             needs_layout_passes=False),
    )
    def sc_pool(srcR, dstR, qu_hbm, dis_hbm, batch_hbm, zeros_hbm, out_hbm,
                srcb, dstb, srcx, dstx, qu_v, dis_v, bat_v, bk, ck, isem):
        cid = lax.axis_index("c")
        sid = lax.axis_index("s")
        wid = cid * _NS + sid
        er0 = cid * per_sc + sid * q
        xrow = cid * per_sc + _NS * q + sid
        pltpu.async_copy(srcR.at[pl.ds(er0, q), :], srcb, isem)
        pltpu.async_copy(dstR.at[pl.ds(er0, q), :], dstb, isem)
        pltpu.async_copy(qu_hbm, qu_v, isem)
        pltpu.async_copy(dis_hbm, dis_v, isem)
        pltpu.async_copy(batch_hbm, bat_v, isem)

        @pl.when(sid < r)
        def _():
            pltpu.async_copy(srcR.at[pl.ds(xrow, 1), :], srcx, isem)
            pltpu.async_copy(dstR.at[pl.ds(xrow, 1), :], dstx, isem)

        pltpu.sync_copy(zeros_hbm, bk)
        pltpu.sync_copy(zeros_hbm, ck)
        pltpu.make_async_copy(srcR.at[pl.ds(er0, q), :], srcb, isem).wait()
        pltpu.make_async_copy(dstR.at[pl.ds(er0, q), :], dstb, isem).wait()
        pltpu.make_async_copy(qu_hbm, qu_v, isem).wait()
        pltpu.make_async_copy(dis_hbm, dis_v, isem).wait()
        pltpu.make_async_copy(batch_hbm, bat_v, isem).wait()

        @pl.when(sid < r)
        def _():
            pltpu.make_async_copy(srcR.at[pl.ds(xrow, 1), :], srcx,
                                  isem).wait()
            pltpu.make_async_copy(dstR.at[pl.ds(xrow, 1), :], dstx,
                                  isem).wait()

        lane = jax.lax.broadcasted_iota(jnp.int32, (16,), 0)

        def edge_row(ref, er):
            for k in range(_CHUNK // 16):
                s16 = ref[0][er, pl.ds(16 * k, 16)]
                d16 = ref[1][er, pl.ds(16 * k, 16)]
                sv = plsc.load_gather(qu_v, [s16])
                dv = plsc.load_gather(dis_v, [d16])
                bv = plsc.load_gather(bat_v, [d16])
                plsc.addupdate_scatter(bk, [lane, bv], sv * dv)

        def erow_loop(er, carry):
            edge_row((srcb, dstb), er)
            return carry

        lax.fori_loop(0, q, erow_loop, 0)

        @pl.when(sid < r)
        def _():
            edge_row((srcx, dstx), 0)

        node0 = wid * npw
        ones16 = jnp.ones((16,), jnp.float32)

        def node_chunk(j, carry):
            idx = node0 + j * 16 + lane
            m = idx < n
            idxc = jnp.minimum(idx, n - 1)
            sv = plsc.load_gather(qu_v, [idxc])
            dv = plsc.load_gather(dis_v, [idxc])
            bv = plsc.load_gather(bat_v, [idxc])
            plsc.addupdate_scatter(bk, [lane, bv], sv * dv, mask=m)
            plsc.addupdate_scatter(ck, [lane, bv], ones16, mask=m)
            return carry

        lax.fori_loop(0, nchunks, node_chunk, 0)

        pltpu.sync_copy(bk, out_hbm.at[pl.ds(wid * 16, 16), :])
        pltpu.sync_copy(ck, out_hbm.at[pl.ds((_NC * _NS + wid) * 16, 16), :])

    return sc_pool


def _tc_a_body(x_ref, w1_ref, d0_ref, d1_ref, o_ref):
    h = jnp.dot(x_ref[...], w1_ref[...], preferred_element_type=jnp.float32)
    deg = d0_ref[...] + d1_ref[...] + 1.0
    o_ref[...] = h * lax.rsqrt(deg)


def _tc_b_body(a0_ref, a1_ref, d0_ref, d1_ref, b1_ref, w2_ref,
               linw_ref, b2_ref, linb_ref, qu_ref, dis_ref, c_ref):
    deg = d0_ref[...] + d1_ref[...] + 1.0
    dis = lax.rsqrt(deg)
    out1 = jax.nn.relu(dis * (a0_ref[...] + a1_ref[...]) + b1_ref[...])
    w = jnp.dot(w2_ref[...], linw_ref[...], preferred_element_type=jnp.float32)
    q = jnp.dot(out1, w, preferred_element_type=jnp.float32)
    qu_ref[...] = dis * q
    dis_ref[...] = dis
    c_ref[...] = jnp.dot(b2_ref[...], linw_ref[...],
                         preferred_element_type=jnp.float32) + linb_ref[...]


def kernel(x, edge_index, edge_attr, batch, W1, b1, W2, b2, lin_W, lin_b):
    n = x.shape[0]
    e = edge_index.shape[1]
    din = x.shape[1]
    h = W1.shape[1]
    srcR = edge_index[0].reshape(e // _CHUNK, _CHUNK)
    dstR = edge_index[1].reshape(e // _CHUNK, _CHUNK)
    rpt = -(-n // (8 * _NS)) * 8
    npad = rpt * _NS

    ones1 = jnp.ones((_CHUNK,), jnp.float32)
    zeros1 = jnp.zeros((rpt,), jnp.float32)
    zerosh = jnp.zeros((rpt, h), jnp.float32)

    degp = _make_sc_pass(npad, e, 1, gather=False)(dstR, ones1, zeros1)
    d0p = degp[:npad].reshape(npad, 1)
    d1p = degp[npad:].reshape(npad, 1)

    xp = jnp.zeros((npad, din), x.dtype).at[:n].set(x)
    u1 = pl.pallas_call(
        _tc_a_body,
        grid=(_NS,),
        in_specs=[
            pl.BlockSpec((rpt, din), lambda i: (i, 0)),
            pl.BlockSpec((din, h), lambda i: (0, 0)),
            pl.BlockSpec((rpt, 1), lambda i: (i, 0)),
            pl.BlockSpec((rpt, 1), lambda i: (i, 0)),
        ],
        out_specs=pl.BlockSpec((rpt, h), lambda i: (i, 0)),
        out_shape=jax.ShapeDtypeStruct((npad, h), jnp.float32),
    )(xp, W1, d0p, d1p)

    aggp = _make_sc_pass(npad, e, h, gather=True)(srcR, dstR, u1, zerosh)

    rb = 2000
    qu, dis, const = pl.pallas_call(
        _tc_b_body,
        grid=(n // rb,),
        in_specs=[
            pl.BlockSpec((rb, h), lambda i: (i, 0)),
            pl.BlockSpec((rb, h), lambda i: (i, 0)),
            pl.BlockSpec((rb, 1), lambda i: (i, 0)),
            pl.BlockSpec((rb, 1), lambda i: (i, 0)),
            pl.BlockSpec((1, h), lambda i: (0, 0)),
            pl.BlockSpec((h, h), lambda i: (0, 0)),
            pl.BlockSpec((h, 1), lambda i: (0, 0)),
            pl.BlockSpec((1, h), lambda i: (0, 0)),
            pl.BlockSpec((1, 1), lambda i: (0, 0)),
        ],
        out_specs=[
            pl.BlockSpec((rb, 1), lambda i: (i, 0)),
            pl.BlockSpec((rb, 1), lambda i: (i, 0)),
            pl.BlockSpec((1, 1), lambda i: (0, 0)),
        ],
        out_shape=[
            jax.ShapeDtypeStruct((n, 1), jnp.float32),
            jax.ShapeDtypeStruct((n, 1), jnp.float32),
            jax.ShapeDtypeStruct((1, 1), jnp.float32),
        ],
    )(aggp[:n], aggp[npad:npad + n], d0p[:n], d1p[:n], b1.reshape(1, h),
      W2, lin_W, b2.reshape(1, h), lin_b.reshape(1, 1))

    zeros16g = jnp.zeros((16, _G), jnp.float32)
    buckets = _make_sc_pool(n, e)(srcR, dstR, qu.reshape(n), dis.reshape(n),
                                  batch, zeros16g)
    nw = _NC * _NS
    sums = jnp.sum(buckets[:16 * nw].reshape(nw * 16, _G), axis=0)
    cnt = jnp.sum(buckets[16 * nw:].reshape(nw * 16, _G), axis=0)
    return jnp.where(cnt > 0, sums / jnp.maximum(cnt, 1.0) + const[0, 0],
                     lin_b[0])


# speedup vs baseline: 1.1462x; 1.1462x over previous
import functools

import jax
import jax.numpy as jnp
from jax import lax
from jax.experimental import pallas as pl
from jax.experimental.pallas import tpu as pltpu
from jax.experimental.pallas import tpu_sc as plsc

_NC = 2
_NS = 16
_CHUNK = 128
_G = 64


def _sc_mesh():
    return plsc.VectorSubcoreMesh(core_axis_name="c", subcore_axis_name="s")


def _make_sc_pass(n, e, d, gather):
    erows = e // _CHUNK
    per_sc = erows // _NC
    q = per_sc // _NS
    r = per_sc % _NS
    nb = next(b for b in (6, 5, 4, 3, 2, 1) if q % b == 0)
    ng = q // nb
    rpt = n // _NS

    scratch = []
    if gather:
        scratch += [pltpu.VMEM((q, _CHUNK), jnp.int32),
                    pltpu.VMEM((1, _CHUNK), jnp.int32)]
    scratch += [
        pltpu.VMEM((q, _CHUNK), jnp.int32),
        pltpu.VMEM((1, _CHUNK), jnp.int32),
    ]
    if gather:
        scratch += [pltpu.VMEM((nb, _CHUNK, d), jnp.float32),
                    pltpu.VMEM_SHARED((n, d), jnp.float32)]
    else:
        scratch += [pltpu.VMEM((_CHUNK,), jnp.float32),
                    pltpu.VMEM_SHARED((n,), jnp.float32)]
    scratch += [pltpu.SemaphoreType.DMA]
    scratch += [pltpu.SemaphoreType.DMA] * (2 * nb if gather else nb)

    out_type = jax.ShapeDtypeStruct(
        (_NC * n, d) if gather else (_NC * n,), jnp.float32)
    params = pltpu.CompilerParams(use_tc_tiling_on_sc=False)

    if gather:
        @functools.partial(pl.kernel, out_type=out_type, mesh=_sc_mesh(),
                           scratch_types=scratch, compiler_params=params)
        def sc_pass(edgeR, vals, zeros, out_hbm,
                    srcb, srcx, dstb, dstx, rows, agg, isem, *sems):
            nv = vals.shape[0]
            ft = nv // rpt
            st = nv - ft * rpt
            gsem, ssem = sems[:nb], sems[nb:]
            cid = lax.axis_index("c")
            sid = lax.axis_index("s")
            row0 = sid * rpt
            er0 = cid * per_sc + sid * q
            xrow = cid * per_sc + _NS * q + sid
            pltpu.async_copy(edgeR.at[pl.ds(er0, q), :], srcb, isem)
            pltpu.async_copy(edgeR.at[pl.ds(erows + er0, q), :], dstb, isem)

            @pl.when(sid < r)
            def _():
                pltpu.async_copy(edgeR.at[pl.ds(xrow, 1), :], srcx, isem)
                pltpu.async_copy(edgeR.at[pl.ds(erows + xrow, 1), :], dstx,
                                 isem)

            @pl.when(jnp.logical_and(cid == 0, sid < ft))
            def _():
                pltpu.sync_copy(vals.at[pl.ds(row0, rpt), :],
                                agg.at[pl.ds(row0, rpt), :])

            if st:
                @pl.when(jnp.logical_and(cid == 0, sid == ft))
                def _():
                    pltpu.sync_copy(vals.at[pl.ds(ft * rpt, st), :],
                                    agg.at[pl.ds(row0, st), :])
                    pltpu.sync_copy(zeros.at[pl.ds(0, rpt - st), :],
                                    agg.at[pl.ds(row0 + st, rpt - st), :])

            @pl.when(jnp.logical_and(cid == 0, sid > ft))
            def _():
                pltpu.sync_copy(zeros, agg.at[pl.ds(row0, rpt), :])

            @pl.when(cid != 0)
            def _():
                pltpu.sync_copy(zeros, agg.at[pl.ds(row0, rpt), :])
            pltpu.make_async_copy(edgeR.at[pl.ds(er0, q), :], srcb,
                                  isem).wait()
            pltpu.make_async_copy(edgeR.at[pl.ds(erows + er0, q), :], dstb,
                                  isem).wait()

            @pl.when(sid < r)
            def _():
                pltpu.make_async_copy(edgeR.at[pl.ds(xrow, 1), :], srcx,
                                      isem).wait()
                pltpu.make_async_copy(edgeR.at[pl.ds(erows + xrow, 1), :],
                                      dstx, isem).wait()

            for u in range(nb):
                pltpu.async_copy(vals.at[srcb.at[u]], rows.at[u], gsem[u])
            plsc.subcore_barrier()

            def group(jj, carry):
                for u in range(nb):
                    pltpu.make_async_copy(vals.at[srcb.at[jj * nb + u]],
                                          rows.at[u], gsem[u]).wait()
                    pltpu.async_copy(rows.at[u], agg.at[dstb.at[jj * nb + u]],
                                     ssem[u], add=True)

                @pl.when(jj < ng - 1)
                def _():
                    for u in range(nb):
                        pltpu.make_async_copy(
                            rows.at[u], agg.at[dstb.at[jj * nb + u]],
                            ssem[u]).wait()
                        pltpu.async_copy(vals.at[srcb.at[(jj + 1) * nb + u]],
                                         rows.at[u], gsem[u])
                return carry

            lax.fori_loop(0, ng, group, 0)
            for u in range(nb):
                pltpu.make_async_copy(rows.at[u],
                                      agg.at[dstb.at[(ng - 1) * nb + u]],
                                      ssem[u]).wait()

            @pl.when(sid < r)
            def _():
                pltpu.async_copy(vals.at[srcx.at[0]], rows.at[0],
                                 gsem[0]).wait()
                pltpu.sync_copy(rows.at[0], agg.at[dstx.at[0]], add=True)

            plsc.subcore_barrier()
            pltpu.sync_copy(agg.at[pl.ds(row0, rpt), :],
                            out_hbm.at[pl.ds(cid * n + row0, rpt), :])
    else:
        @functools.partial(pl.kernel, out_type=out_type, mesh=_sc_mesh(),
                           scratch_types=scratch, compiler_params=params)
        def sc_pass(edgeR, ones_hbm, zeros_hbm, out_hbm,
                    dstb, dstx, ones_v, agg, isem, *ssem):
            cid = lax.axis_index("c")
            sid = lax.axis_index("s")
            row0 = sid * rpt
            er0 = erows + cid * per_sc + sid * q
            xrow = erows + cid * per_sc + _NS * q + sid
            pltpu.async_copy(edgeR.at[pl.ds(er0, q), :], dstb, isem)

            @pl.when(sid < r)
            def _():
                pltpu.async_copy(edgeR.at[pl.ds(xrow, 1), :], dstx, isem)

            pltpu.sync_copy(ones_hbm, ones_v)
            pltpu.sync_copy(zeros_hbm, agg.at[pl.ds(row0, rpt)])
            pltpu.make_async_copy(edgeR.at[pl.ds(er0, q), :], dstb,
                                  isem).wait()

            @pl.when(sid < r)
            def _():
                pltpu.make_async_copy(edgeR.at[pl.ds(xrow, 1), :], dstx,
                                      isem).wait()

            plsc.subcore_barrier()

            def group(jj, carry):
                @pl.when(jj > 0)
                def _():
                    for u in range(nb):
                        pltpu.make_async_copy(
                            ones_v, agg.at[dstb.at[(jj - 1) * nb + u]],
                            ssem[u]).wait()
                for u in range(nb):
                    pltpu.async_copy(ones_v, agg.at[dstb.at[jj * nb + u]],
                                     ssem[u], add=True)
                return carry

            lax.fori_loop(0, ng, group, 0)
            for u in range(nb):
                pltpu.make_async_copy(ones_v,
                                      agg.at[dstb.at[(ng - 1) * nb + u]],
                                      ssem[u]).wait()

            @pl.when(sid < r)
            def _():
                pltpu.sync_copy(ones_v, agg.at[dstx.at[0]], add=True)

            plsc.subcore_barrier()
            pltpu.sync_copy(agg.at[pl.ds(row0, rpt)],
                            out_hbm.at[pl.ds(cid * n + row0, rpt)])

    return sc_pass


def _make_sc_pool(n, e):
    erows = e // _CHUNK
    per_sc = erows // _NC
    q = per_sc // _NS
    r = per_sc % _NS
    npw = -(-n // (_NC * _NS * 16)) * 16
    nchunks = npw // 16

    scratch = [
        pltpu.VMEM((q, _CHUNK), jnp.int32),
        pltpu.VMEM((q, _CHUNK), jnp.int32),
        pltpu.VMEM((1, _CHUNK), jnp.int32),
        pltpu.VMEM((1, _CHUNK), jnp.int32),
        pltpu.VMEM((n,), jnp.float32),
        pltpu.VMEM((n,), jnp.float32),
        pltpu.VMEM((n,), jnp.int32),
        pltpu.VMEM((16, _G), jnp.float32),
        pltpu.VMEM((16, _G), jnp.float32),
        pltpu.VMEM((16,), jnp.int32),
        pltpu.VMEM((16,), jnp.int32),
        pltpu.VMEM_SHARED((32, _G), jnp.float32),
        pltpu.SemaphoreType.DMA,
    ]

    @functools.partial(
        pl.kernel,
        out_type=jax.ShapeDtypeStruct((_NC * 32, _G), jnp.float32),
        mesh=_sc_mesh(),
        scratch_types=scratch,
        compiler_params=pltpu.CompilerParams(use_tc_tiling_on_sc=False,
                                             needs_layout_passes=False),
    )
    def sc_pool(edgeR, qu_hbm, dis_hbm, batch_hbm, zeros_hbm, out_hbm,
                srcb, dstb, srcx, dstx, qu_v, dis_v, bat_v, bk, ck,
                rid, rid2, shared, isem):
        cid = lax.axis_index("c")
        sid = lax.axis_index("s")
        er0 = cid * per_sc + sid * q
        xrow = cid * per_sc + _NS * q + sid
        pltpu.async_copy(edgeR.at[pl.ds(er0, q), :], srcb, isem)
        pltpu.async_copy(edgeR.at[pl.ds(erows + er0, q), :], dstb, isem)
        pltpu.async_copy(qu_hbm, qu_v, isem)
        pltpu.async_copy(dis_hbm, dis_v, isem)
        pltpu.async_copy(batch_hbm, bat_v, isem)

        @pl.when(sid < r)
        def _():
            pltpu.async_copy(edgeR.at[pl.ds(xrow, 1), :], srcx, isem)
            pltpu.async_copy(edgeR.at[pl.ds(erows + xrow, 1), :], dstx, isem)

        pltpu.sync_copy(zeros_hbm, bk)
        pltpu.sync_copy(zeros_hbm, ck)

        @pl.when(sid == 0)
        def _():
            pltpu.sync_copy(zeros_hbm, shared.at[pl.ds(0, 16), :])
            pltpu.sync_copy(zeros_hbm, shared.at[pl.ds(16, 16), :])

        lane16 = jax.lax.broadcasted_iota(jnp.int32, (16,), 0)
        rid[...] = lane16
        rid2[...] = lane16 + 16
        pltpu.make_async_copy(edgeR.at[pl.ds(er0, q), :], srcb, isem).wait()
        pltpu.make_async_copy(edgeR.at[pl.ds(erows + er0, q), :], dstb,
                              isem).wait()
        pltpu.make_async_copy(qu_hbm, qu_v, isem).wait()
        pltpu.make_async_copy(dis_hbm, dis_v, isem).wait()
        pltpu.make_async_copy(batch_hbm, bat_v, isem).wait()

        @pl.when(sid < r)
        def _():
            pltpu.make_async_copy(edgeR.at[pl.ds(xrow, 1), :], srcx,
                                  isem).wait()
            pltpu.make_async_copy(edgeR.at[pl.ds(erows + xrow, 1), :], dstx,
                                  isem).wait()

        plsc.subcore_barrier()

        lane = jax.lax.broadcasted_iota(jnp.int32, (16,), 0)

        def edge_row(ref, er):
            for k in range(_CHUNK // 16):
                s16 = ref[0][er, pl.ds(16 * k, 16)]
                d16 = ref[1][er, pl.ds(16 * k, 16)]
                sv = plsc.load_gather(qu_v, [s16])
                dv = plsc.load_gather(dis_v, [d16])
                bv = plsc.load_gather(bat_v, [d16])
                plsc.addupdate_scatter(bk, [lane, bv], sv * dv)

        def erow_loop(er, carry):
            edge_row((srcb, dstb), er)
            return carry

        lax.fori_loop(0, q, erow_loop, 0)

        @pl.when(sid < r)
        def _():
            edge_row((srcx, dstx), 0)

        node0 = (cid * _NS + sid) * npw
        ones16 = jnp.ones((16,), jnp.float32)

        def node_chunk(j, carry):
            idx = node0 + j * 16 + lane
            m = idx < n
            idxc = jnp.minimum(idx, n - 1)
            sv = plsc.load_gather(qu_v, [idxc])
            dv = plsc.load_gather(dis_v, [idxc])
            bv = plsc.load_gather(bat_v, [idxc])
            plsc.addupdate_scatter(bk, [lane, bv], sv * dv, mask=m)
            plsc.addupdate_scatter(ck, [lane, bv], ones16, mask=m)
            return carry

        lax.fori_loop(0, nchunks, node_chunk, 0)

        pltpu.sync_copy(bk, shared.at[rid], add=True)
        pltpu.sync_copy(ck, shared.at[rid2], add=True)
        plsc.subcore_barrier()

        @pl.when(sid == 0)
        def _():
            pltpu.sync_copy(shared, out_hbm.at[pl.ds(cid * 32, 32), :])

    return sc_pool


def _tc_a_body(x_ref, w1_ref, d0_ref, d1_ref, o_ref):
    h = jnp.dot(x_ref[...], w1_ref[...], preferred_element_type=jnp.float32)
    deg = d0_ref[...] + d1_ref[...] + 1.0
    o_ref[...] = h * lax.rsqrt(deg)


def _tc_b_body(a0_ref, a1_ref, d0_ref, d1_ref, b1_ref, w2_ref,
               linw_ref, b2_ref, linb_ref, qu_ref, dis_ref, c_ref):
    deg = d0_ref[...] + d1_ref[...] + 1.0
    dis = lax.rsqrt(deg)
    out1 = jax.nn.relu(dis * (a0_ref[0] + a1_ref[0]) + b1_ref[...])
    w = jnp.dot(w2_ref[...], linw_ref[...], preferred_element_type=jnp.float32)
    q = jnp.dot(out1, w, preferred_element_type=jnp.float32)
    qu_ref[...] = dis * q
    dis_ref[...] = dis
    c_ref[...] = jnp.dot(b2_ref[...], linw_ref[...],
                         preferred_element_type=jnp.float32) + linb_ref[...]


def kernel(x, edge_index, edge_attr, batch, W1, b1, W2, b2, lin_W, lin_b):
    n = x.shape[0]
    e = edge_index.shape[1]
    din = x.shape[1]
    h = W1.shape[1]
    erows = e // _CHUNK
    edgeR = edge_index.reshape(2 * erows, _CHUNK)
    rpt = -(-n // (8 * _NS)) * 8
    npad = rpt * _NS

    ones1 = jnp.ones((_CHUNK,), jnp.float32)
    zeros1 = jnp.zeros((rpt,), jnp.float32)
    zerosh = jnp.zeros((rpt, h), jnp.float32)

    degp = _make_sc_pass(npad, e, 1, gather=False)(edgeR, ones1, zeros1)
    d0p = degp[:npad].reshape(npad, 1)
    d1p = degp[npad:].reshape(npad, 1)

    rb = 2000
    u1 = pl.pallas_call(
        _tc_a_body,
        grid=(n // rb,),
        in_specs=[
            pl.BlockSpec((rb, din), lambda i: (i, 0)),
            pl.BlockSpec((din, h), lambda i: (0, 0)),
            pl.BlockSpec((rb, 1), lambda i: (i, 0)),
            pl.BlockSpec((rb, 1), lambda i: (i, 0)),
        ],
        out_specs=pl.BlockSpec((rb, h), lambda i: (i, 0)),
        out_shape=jax.ShapeDtypeStruct((n, h), jnp.float32),
    )(x, W1, d0p[:n], d1p[:n])

    aggp = _make_sc_pass(npad, e, h, gather=True)(edgeR, u1, zerosh)
    aggp3 = aggp.reshape(_NC, npad, h)

    qu, dis, const = pl.pallas_call(
        _tc_b_body,
        grid=(n // rb,),
        in_specs=[
            pl.BlockSpec((1, rb, h), lambda i: (0, i, 0)),
            pl.BlockSpec((1, rb, h), lambda i: (1, i, 0)),
            pl.BlockSpec((rb, 1), lambda i: (i, 0)),
            pl.BlockSpec((rb, 1), lambda i: (i, 0)),
            pl.BlockSpec((1, h), lambda i: (0, 0)),
            pl.BlockSpec((h, h), lambda i: (0, 0)),
            pl.BlockSpec((h, 1), lambda i: (0, 0)),
            pl.BlockSpec((1, h), lambda i: (0, 0)),
            pl.BlockSpec((1, 1), lambda i: (0, 0)),
        ],
        out_specs=[
            pl.BlockSpec((rb, 1), lambda i: (i, 0)),
            pl.BlockSpec((rb, 1), lambda i: (i, 0)),
            pl.BlockSpec((1, 1), lambda i: (0, 0)),
        ],
        out_shape=[
            jax.ShapeDtypeStruct((n, 1), jnp.float32),
            jax.ShapeDtypeStruct((n, 1), jnp.float32),
            jax.ShapeDtypeStruct((1, 1), jnp.float32),
        ],
    )(aggp3, aggp3, d0p[:n], d1p[:n], b1.reshape(1, h),
      W2, lin_W, b2.reshape(1, h), lin_b.reshape(1, 1))

    zeros16g = jnp.zeros((16, _G), jnp.float32)
    buckets = _make_sc_pool(n, e)(edgeR, qu.reshape(n), dis.reshape(n),
                                  batch, zeros16g)
    bsc = buckets.reshape(_NC, 2, 16, _G)
    sums = jnp.sum(bsc[:, 0], axis=(0, 1))
    cnt = jnp.sum(bsc[:, 1], axis=(0, 1))
    return jnp.where(cnt > 0, sums / jnp.maximum(cnt, 1.0) + const[0, 0],
                     lin_b[0])


# speedup vs baseline: 1.3095x; 1.1424x over previous
import functools

import jax
import jax.numpy as jnp
from jax import lax
from jax.experimental import pallas as pl
from jax.experimental.pallas import tpu as pltpu
from jax.experimental.pallas import tpu_sc as plsc

_NC = 2
_NS = 16
_CHUNK = 128
_G = 64


def _sc_mesh():
    return plsc.VectorSubcoreMesh(core_axis_name="c", subcore_axis_name="s")


def _make_sc_pass(n, e, d, gather):
    erows = e // _CHUNK
    per_sc = erows // _NC
    q = per_sc // _NS
    r = per_sc % _NS
    nb = next(b for b in (6, 5, 4, 3, 2, 1) if q % b == 0)
    ng = q // nb
    rpt = n // _NS

    scratch = []
    if gather:
        scratch += [pltpu.VMEM((q, _CHUNK), jnp.int32),
                    pltpu.VMEM((1, _CHUNK), jnp.int32)]
    scratch += [
        pltpu.VMEM((q, _CHUNK), jnp.int32),
        pltpu.VMEM((1, _CHUNK), jnp.int32),
    ]
    if gather:
        scratch += [pltpu.VMEM((nb, _CHUNK, d), jnp.float32),
                    pltpu.VMEM_SHARED((n, d), jnp.float32)]
    else:
        scratch += [pltpu.VMEM((_CHUNK,), jnp.float32),
                    pltpu.VMEM_SHARED((n,), jnp.float32)]
    scratch += [pltpu.SemaphoreType.DMA]
    scratch += [pltpu.SemaphoreType.DMA] * (2 * nb if gather else nb)

    out_type = jax.ShapeDtypeStruct(
        (_NC * n, d) if gather else (_NC * n,), jnp.float32)
    params = pltpu.CompilerParams(use_tc_tiling_on_sc=False)

    if gather:
        @functools.partial(pl.kernel, out_type=out_type, mesh=_sc_mesh(),
                           scratch_types=scratch, compiler_params=params)
        def sc_pass(edgeR, vals, zeros, out_hbm,
                    srcb, srcx, dstb, dstx, rows, agg, isem, *sems):
            nv = vals.shape[0]
            ft = nv // rpt
            st = nv - ft * rpt
            gsem, ssem = sems[:nb], sems[nb:]
            cid = lax.axis_index("c")
            sid = lax.axis_index("s")
            row0 = sid * rpt
            er0 = cid * per_sc + sid * q
            xrow = cid * per_sc + _NS * q + sid
            pltpu.async_copy(edgeR.at[pl.ds(er0, q), :], srcb, isem)
            pltpu.async_copy(edgeR.at[pl.ds(erows + er0, q), :], dstb, isem)

            @pl.when(sid < r)
            def _():
                pltpu.async_copy(edgeR.at[pl.ds(xrow, 1), :], srcx, isem)
                pltpu.async_copy(edgeR.at[pl.ds(erows + xrow, 1), :], dstx,
                                 isem)

            @pl.when(jnp.logical_and(cid == 0, sid < ft))
            def _():
                pltpu.sync_copy(vals.at[pl.ds(row0, rpt), :],
                                agg.at[pl.ds(row0, rpt), :])

            if st:
                @pl.when(jnp.logical_and(cid == 0, sid == ft))
                def _():
                    pltpu.sync_copy(vals.at[pl.ds(ft * rpt, st), :],
                                    agg.at[pl.ds(row0, st), :])
                    pltpu.sync_copy(zeros.at[pl.ds(0, rpt - st), :],
                                    agg.at[pl.ds(row0 + st, rpt - st), :])

            @pl.when(jnp.logical_and(cid == 0, sid > ft))
            def _():
                pltpu.sync_copy(zeros, agg.at[pl.ds(row0, rpt), :])

            @pl.when(cid != 0)
            def _():
                pltpu.sync_copy(zeros, agg.at[pl.ds(row0, rpt), :])
            pltpu.make_async_copy(edgeR.at[pl.ds(er0, q), :], srcb,
                                  isem).wait()
            pltpu.make_async_copy(edgeR.at[pl.ds(erows + er0, q), :], dstb,
                                  isem).wait()

            @pl.when(sid < r)
            def _():
                pltpu.make_async_copy(edgeR.at[pl.ds(xrow, 1), :], srcx,
                                      isem).wait()
                pltpu.make_async_copy(edgeR.at[pl.ds(erows + xrow, 1), :],
                                      dstx, isem).wait()

            for u in range(nb):
                pltpu.async_copy(vals.at[srcb.at[u]], rows.at[u], gsem[u])
            plsc.subcore_barrier()

            def group(jj, carry):
                for u in range(nb):
                    pltpu.make_async_copy(vals.at[srcb.at[jj * nb + u]],
                                          rows.at[u], gsem[u]).wait()
                    pltpu.async_copy(rows.at[u], agg.at[dstb.at[jj * nb + u]],
                                     ssem[u], add=True)

                @pl.when(jj < ng - 1)
                def _():
                    for u in range(nb):
                        pltpu.make_async_copy(
                            rows.at[u], agg.at[dstb.at[jj * nb + u]],
                            ssem[u]).wait()
                        pltpu.async_copy(vals.at[srcb.at[(jj + 1) * nb + u]],
                                         rows.at[u], gsem[u])
                return carry

            lax.fori_loop(0, ng, group, 0)
            for u in range(nb):
                pltpu.make_async_copy(rows.at[u],
                                      agg.at[dstb.at[(ng - 1) * nb + u]],
                                      ssem[u]).wait()

            @pl.when(sid < r)
            def _():
                pltpu.async_copy(vals.at[srcx.at[0]], rows.at[0],
                                 gsem[0]).wait()
                pltpu.sync_copy(rows.at[0], agg.at[dstx.at[0]], add=True)

            plsc.subcore_barrier()
            pltpu.sync_copy(agg.at[pl.ds(row0, rpt), :],
                            out_hbm.at[pl.ds(cid * n + row0, rpt), :])
    else:
        @functools.partial(pl.kernel, out_type=out_type, mesh=_sc_mesh(),
                           scratch_types=scratch, compiler_params=params)
        def sc_pass(edgeR, ones_hbm, zeros_hbm, out_hbm,
                    dstb, dstx, ones_v, agg, isem, *ssem):
            cid = lax.axis_index("c")
            sid = lax.axis_index("s")
            row0 = sid * rpt
            er0 = erows + cid * per_sc + sid * q
            xrow = erows + cid * per_sc + _NS * q + sid
            pltpu.async_copy(edgeR.at[pl.ds(er0, q), :], dstb, isem)

            @pl.when(sid < r)
            def _():
                pltpu.async_copy(edgeR.at[pl.ds(xrow, 1), :], dstx, isem)

            pltpu.sync_copy(ones_hbm, ones_v)
            pltpu.sync_copy(zeros_hbm, agg.at[pl.ds(row0, rpt)])
            pltpu.make_async_copy(edgeR.at[pl.ds(er0, q), :], dstb,
                                  isem).wait()

            @pl.when(sid < r)
            def _():
                pltpu.make_async_copy(edgeR.at[pl.ds(xrow, 1), :], dstx,
                                      isem).wait()

            plsc.subcore_barrier()

            def group(jj, carry):
                @pl.when(jj > 0)
                def _():
                    for u in range(nb):
                        pltpu.make_async_copy(
                            ones_v, agg.at[dstb.at[(jj - 1) * nb + u]],
                            ssem[u]).wait()
                for u in range(nb):
                    pltpu.async_copy(ones_v, agg.at[dstb.at[jj * nb + u]],
                                     ssem[u], add=True)
                return carry

            lax.fori_loop(0, ng, group, 0)
            for u in range(nb):
                pltpu.make_async_copy(ones_v,
                                      agg.at[dstb.at[(ng - 1) * nb + u]],
                                      ssem[u]).wait()

            @pl.when(sid < r)
            def _():
                pltpu.sync_copy(ones_v, agg.at[dstx.at[0]], add=True)

            plsc.subcore_barrier()
            pltpu.sync_copy(agg.at[pl.ds(row0, rpt)],
                            out_hbm.at[pl.ds(cid * n + row0, rpt)])

    return sc_pass


def _make_sc_pool(n, e):
    erows = e // _CHUNK
    per_sc = erows // _NC
    q = per_sc // _NS
    r = per_sc % _NS
    npw = -(-n // (_NC * _NS * 16)) * 16
    nchunks = npw // 16

    scratch = [
        pltpu.VMEM((q, _CHUNK), jnp.int32),
        pltpu.VMEM((q, _CHUNK), jnp.int32),
        pltpu.VMEM((1, _CHUNK), jnp.int32),
        pltpu.VMEM((1, _CHUNK), jnp.int32),
        pltpu.VMEM((n,), jnp.float32),
        pltpu.VMEM((n,), jnp.float32),
        pltpu.VMEM((n,), jnp.int32),
        pltpu.VMEM((16, _G), jnp.float32),
        pltpu.VMEM((16, _G), jnp.float32),
        pltpu.VMEM((16,), jnp.int32),
        pltpu.VMEM((16,), jnp.int32),
        pltpu.VMEM_SHARED((32, _G), jnp.float32),
        pltpu.SemaphoreType.DMA,
    ]

    @functools.partial(
        pl.kernel,
        out_type=jax.ShapeDtypeStruct((_NC * 32, _G), jnp.float32),
        mesh=_sc_mesh(),
        scratch_types=scratch,
        compiler_params=pltpu.CompilerParams(use_tc_tiling_on_sc=False,
                                             needs_layout_passes=False),
    )
    def sc_pool(edgeR, qu_hbm, dis_hbm, batch_hbm, zeros_hbm, out_hbm,
                srcb, dstb, srcx, dstx, qu_v, dis_v, bat_v, bk, ck,
                rid, rid2, shared, isem):
        cid = lax.axis_index("c")
        sid = lax.axis_index("s")
        er0 = cid * per_sc + sid * q
        xrow = cid * per_sc + _NS * q + sid
        pltpu.async_copy(edgeR.at[pl.ds(er0, q), :], srcb, isem)
        pltpu.async_copy(edgeR.at[pl.ds(erows + er0, q), :], dstb, isem)
        pltpu.async_copy(qu_hbm, qu_v, isem)
        pltpu.async_copy(dis_hbm, dis_v, isem)
        pltpu.async_copy(batch_hbm, bat_v, isem)

        @pl.when(sid < r)
        def _():
            pltpu.async_copy(edgeR.at[pl.ds(xrow, 1), :], srcx, isem)
            pltpu.async_copy(edgeR.at[pl.ds(erows + xrow, 1), :], dstx, isem)

        pltpu.sync_copy(zeros_hbm, bk)
        pltpu.sync_copy(zeros_hbm, ck)

        @pl.when(sid == 0)
        def _():
            pltpu.sync_copy(zeros_hbm, shared.at[pl.ds(0, 16), :])
            pltpu.sync_copy(zeros_hbm, shared.at[pl.ds(16, 16), :])

        lane16 = jax.lax.broadcasted_iota(jnp.int32, (16,), 0)
        rid[...] = lane16
        rid2[...] = lane16 + 16
        pltpu.make_async_copy(edgeR.at[pl.ds(er0, q), :], srcb, isem).wait()
        pltpu.make_async_copy(edgeR.at[pl.ds(erows + er0, q), :], dstb,
                              isem).wait()
        pltpu.make_async_copy(qu_hbm, qu_v, isem).wait()
        pltpu.make_async_copy(dis_hbm, dis_v, isem).wait()
        pltpu.make_async_copy(batch_hbm, bat_v, isem).wait()

        @pl.when(sid < r)
        def _():
            pltpu.make_async_copy(edgeR.at[pl.ds(xrow, 1), :], srcx,
                                  isem).wait()
            pltpu.make_async_copy(edgeR.at[pl.ds(erows + xrow, 1), :], dstx,
                                  isem).wait()

        plsc.subcore_barrier()

        lane = jax.lax.broadcasted_iota(jnp.int32, (16,), 0)

        def edge_row(ref, er):
            for k in range(_CHUNK // 16):
                s16 = ref[0][er, pl.ds(16 * k, 16)]
                d16 = ref[1][er, pl.ds(16 * k, 16)]
                sv = plsc.load_gather(qu_v, [s16])
                dv = plsc.load_gather(dis_v, [d16])
                bv = plsc.load_gather(bat_v, [d16])
                plsc.addupdate_scatter(bk, [lane, bv], sv * dv)

        def erow_loop(er, carry):
            edge_row((srcb, dstb), er)
            return carry

        lax.fori_loop(0, q, erow_loop, 0)

        @pl.when(sid < r)
        def _():
            edge_row((srcx, dstx), 0)

        node0 = (cid * _NS + sid) * npw
        ones16 = jnp.ones((16,), jnp.float32)

        def node_chunk(j, carry):
            idx = node0 + j * 16 + lane
            m = idx < n
            idxc = jnp.minimum(idx, n - 1)
            sv = plsc.load_gather(qu_v, [idxc])
            dv = plsc.load_gather(dis_v, [idxc])
            bv = plsc.load_gather(bat_v, [idxc])
            plsc.addupdate_scatter(bk, [lane, bv], sv * dv, mask=m)
            plsc.addupdate_scatter(ck, [lane, bv], ones16, mask=m)
            return carry

        lax.fori_loop(0, nchunks, node_chunk, 0)

        pltpu.sync_copy(bk, shared.at[rid], add=True)
        pltpu.sync_copy(ck, shared.at[rid2], add=True)
        plsc.subcore_barrier()

        @pl.when(sid == 0)
        def _():
            pltpu.sync_copy(shared, out_hbm.at[pl.ds(cid * 32, 32), :])

    return sc_pool


def _tc_a_body(x_ref, w1_ref, d0_ref, d1_ref, o_ref):
    n = x_ref.shape[0]
    h = jnp.dot(x_ref[...], w1_ref[...], preferred_element_type=jnp.float32)
    deg = jnp.transpose(d0_ref[...] + d1_ref[...] + 1.0)[:n]
    o_ref[...] = h * lax.rsqrt(deg)


def _tc_b_body(a0_ref, a1_ref, d0_ref, d1_ref, b1_ref, w2_ref,
               linw_ref, b2_ref, linb_ref, qu_ref, dis_ref, c_ref):
    n = qu_ref.shape[1]
    deg = jnp.transpose(d0_ref[...] + d1_ref[...] + 1.0)[:n]
    dis = lax.rsqrt(deg)
    out1 = jax.nn.relu(dis * (a0_ref[0, :n] + a1_ref[0, :n]) + b1_ref[...])
    w = jnp.dot(w2_ref[...], linw_ref[...], preferred_element_type=jnp.float32)
    q = jnp.dot(out1, w, preferred_element_type=jnp.float32)
    qu_ref[...] = jnp.transpose(dis * q)
    dis_ref[...] = jnp.transpose(dis)
    c_ref[...] = jnp.dot(b2_ref[...], linw_ref[...],
                         preferred_element_type=jnp.float32) + linb_ref[...]


def kernel(x, edge_index, edge_attr, batch, W1, b1, W2, b2, lin_W, lin_b):
    n = x.shape[0]
    e = edge_index.shape[1]
    din = x.shape[1]
    h = W1.shape[1]
    erows = e // _CHUNK
    edgeR = edge_index.reshape(2 * erows, _CHUNK)
    rpt = -(-n // (8 * _NS)) * 8
    npad = rpt * _NS

    ones1 = jnp.ones((_CHUNK,), jnp.float32)
    zeros1 = jnp.zeros((rpt,), jnp.float32)
    zerosh = jnp.zeros((rpt, h), jnp.float32)

    degp = _make_sc_pass(npad, e, 1, gather=False)(edgeR, ones1, zeros1)
    d0r = degp[:npad].reshape(1, npad)
    d1r = degp[npad:].reshape(1, npad)

    u1 = pl.pallas_call(
        _tc_a_body,
        out_shape=jax.ShapeDtypeStruct((n, h), jnp.float32),
    )(x, W1, d0r, d1r)

    aggp = _make_sc_pass(npad, e, h, gather=True)(edgeR, u1, zerosh)
    aggp3 = aggp.reshape(_NC, npad, h)

    qu, dis, const = pl.pallas_call(
        _tc_b_body,
        grid=(1,),
        in_specs=[
            pl.BlockSpec((1, npad, h), lambda i: (0, 0, 0)),
            pl.BlockSpec((1, npad, h), lambda i: (1, 0, 0)),
            pl.BlockSpec((1, npad), lambda i: (0, 0)),
            pl.BlockSpec((1, npad), lambda i: (0, 0)),
            pl.BlockSpec((1, h), lambda i: (0, 0)),
            pl.BlockSpec((h, h), lambda i: (0, 0)),
            pl.BlockSpec((h, 1), lambda i: (0, 0)),
            pl.BlockSpec((1, h), lambda i: (0, 0)),
            pl.BlockSpec((1, 1), lambda i: (0, 0)),
        ],
        out_specs=[
            pl.BlockSpec((1, n), lambda i: (0, 0)),
            pl.BlockSpec((1, n), lambda i: (0, 0)),
            pl.BlockSpec((1, 1), lambda i: (0, 0)),
        ],
        out_shape=[
            jax.ShapeDtypeStruct((1, n), jnp.float32),
            jax.ShapeDtypeStruct((1, n), jnp.float32),
            jax.ShapeDtypeStruct((1, 1), jnp.float32),
        ],
    )(aggp3, aggp3, d0r, d1r, b1.reshape(1, h),
      W2, lin_W, b2.reshape(1, h), lin_b.reshape(1, 1))

    zeros16g = jnp.zeros((16, _G), jnp.float32)
    buckets = _make_sc_pool(n, e)(edgeR, qu.reshape(n), dis.reshape(n),
                                  batch, zeros16g)
    bsc = buckets.reshape(_NC, 2, 16, _G)
    sums = jnp.sum(bsc[:, 0], axis=(0, 1))
    cnt = jnp.sum(bsc[:, 1], axis=(0, 1))
    return jnp.where(cnt > 0, sums / jnp.maximum(cnt, 1.0) + const[0, 0],
                     lin_b[0])
